# Initial kernel scaffold; baseline (speedup 1.0000x reference)
import jax
import jax.numpy as jnp
from jax.experimental import pallas as pl


def kernel(x, balanced_bias, num_assigned_tokens):
    raise NotImplementedError("write your pallas kernel here")


# speedup vs baseline: 7.2755x; 7.2755x over previous
import jax
import jax.numpy as jnp
from jax.experimental import pallas as pl

_HIDDEN = 4096
_TOPK = 8
_BANK = 64
_NGROUPS = _HIDDEN // _BANK

_TOKENS_PER_BLOCK = 256


def _body(x_ref, bias_ref, nat_ref, out_ref, counts_ref):
    i = pl.program_id(0)
    xb = x_ref[...]
    scores = jnp.abs(xb) + bias_ref[...][None, :, :]
    s = scores
    neg = jnp.float32(-jnp.inf)
    for _ in range(_TOPK - 1):
        m = jnp.max(s, axis=-1, keepdims=True)
        s = jnp.where(s >= m, neg, s)
    t = jnp.max(s, axis=-1, keepdims=True)
    mask = scores >= t
    out = jnp.where(mask, xb, jnp.float32(0.0))
    out_ref[...] = out
    partial = jnp.sum((out != 0.0).astype(jnp.float32), axis=0)

    @pl.when(i == 0)
    def _init():
        counts_ref[...] = nat_ref[...] + partial

    @pl.when(i != 0)
    def _acc():
        counts_ref[...] += partial


def kernel(x, balanced_bias, num_assigned_tokens):
    n_tokens = x.shape[0] * x.shape[1]
    xv = x.reshape(n_tokens, _NGROUPS, _BANK)
    biasv = balanced_bias.reshape(_NGROUPS, _BANK)
    natv = num_assigned_tokens.reshape(_NGROUPS, _BANK)
    grid = n_tokens // _TOKENS_PER_BLOCK

    out, counts = pl.pallas_call(
        _body,
        grid=(grid,),
        in_specs=[
            pl.BlockSpec((_TOKENS_PER_BLOCK, _NGROUPS, _BANK),
                         lambda i: (i, 0, 0)),
            pl.BlockSpec((_NGROUPS, _BANK), lambda i: (0, 0)),
            pl.BlockSpec((_NGROUPS, _BANK), lambda i: (0, 0)),
        ],
        out_specs=[
            pl.BlockSpec((_TOKENS_PER_BLOCK, _NGROUPS, _BANK),
                         lambda i: (i, 0, 0)),
            pl.BlockSpec((_NGROUPS, _BANK), lambda i: (0, 0)),
        ],
        out_shape=[
            jax.ShapeDtypeStruct((n_tokens, _NGROUPS, _BANK), jnp.float32),
            jax.ShapeDtypeStruct((_NGROUPS, _BANK), jnp.float32),
        ],
    )(xv, biasv, natv)

    return out.reshape(x.shape), counts.reshape(_HIDDEN)
